# baseline (speedup 1.0000x reference)
import functools

import jax
import jax.numpy as jnp
from jax import lax
from jax.experimental import pallas as pl
from jax.experimental.pallas import tpu as pltpu
from jax.experimental.pallas import tpu_sc as plsc

B = 16384
E = 64
NC = 2
NS = 16
NW = NC * NS
BPW = B // NW
CHUNK = 128
NCHUNK = BPW // CHUNK


def _sc_gather(uidx, bidx, user_table, book_table):
    mesh = plsc.VectorSubcoreMesh(core_axis_name="c", subcore_axis_name="s")

    @functools.partial(
        pl.kernel,
        out_type=(
            jax.ShapeDtypeStruct((B, E), jnp.float32),
            jax.ShapeDtypeStruct((B, E), jnp.float32),
        ),
        mesh=mesh,
        compiler_params=pltpu.CompilerParams(use_tc_tiling_on_sc=False),
        scratch_types=[
            pltpu.VMEM((NCHUNK, CHUNK), jnp.int32),
            pltpu.VMEM((NCHUNK, CHUNK), jnp.int32),
            pltpu.VMEM((BPW, E), jnp.float32),
            pltpu.VMEM((BPW, E), jnp.float32),
            pltpu.SemaphoreType.DMA,
        ],
    )
    def body(uidx_hbm, bidx_hbm, utab_hbm, btab_hbm, uout_hbm, bout_hbm,
             uidx_v, bidx_v, urows_v, brows_v, sem):
        wid = lax.axis_index("s") * NC + lax.axis_index("c")
        base = wid * BPW
        pltpu.sync_copy(uidx_hbm.at[wid], uidx_v)
        pltpu.sync_copy(bidx_hbm.at[wid], bidx_v)
        copies = []
        for j in range(NCHUNK):
            copies.append(pltpu.async_copy(
                utab_hbm.at[uidx_v.at[j]],
                urows_v.at[pl.ds(j * CHUNK, CHUNK)], sem))
            copies.append(pltpu.async_copy(
                btab_hbm.at[bidx_v.at[j]],
                brows_v.at[pl.ds(j * CHUNK, CHUNK)], sem))
        for c in copies:
            c.wait()
        pltpu.sync_copy(urows_v, uout_hbm.at[pl.ds(base, BPW)])
        pltpu.sync_copy(brows_v, bout_hbm.at[pl.ds(base, BPW)])

    return body(uidx, bidx, user_table, book_table)


def _mlp_body(u_ref, b_ref, w1_ref, b1_ref, w2_ref, b2_ref, w3_ref, b3_ref,
              w4_ref, b4_ref, w5_ref, b5_ref, out_ref):
    u = u_ref[...]
    b = b_ref[...]
    w1 = w1_ref[...]
    hp = jax.lax.Precision.HIGHEST
    h = jnp.dot(u, w1[:E, :], preferred_element_type=jnp.float32, precision=hp)
    h += jnp.dot(b, w1[E:, :], preferred_element_type=jnp.float32, precision=hp)
    h = jax.nn.sigmoid(h + b1_ref[...])
    h = jax.nn.sigmoid(
        jnp.dot(h, w2_ref[...], preferred_element_type=jnp.float32, precision=hp)
        + b2_ref[...])
    h = jnp.tanh(
        jnp.dot(h, w3_ref[...], preferred_element_type=jnp.float32, precision=hp)
        + b3_ref[...])
    h = jnp.tanh(
        jnp.dot(h, w4_ref[...], preferred_element_type=jnp.float32, precision=hp)
        + b4_ref[...])
    w5 = w5_ref[...]
    out = jnp.dot(u * b, w5[:E, :], preferred_element_type=jnp.float32, precision=hp)
    out += jnp.dot(h, w5[E:, :], preferred_element_type=jnp.float32, precision=hp)
    out_ref[...] = out + b5_ref[...]


def _mlp(uvec, bvec, W1, b1, W2, b2, W3, b3, W4, b4, W5, b5, block=2048):
    nblk = B // block
    full = lambda shape: pl.BlockSpec(shape, lambda i: (0, 0))
    return pl.pallas_call(
        _mlp_body,
        grid=(nblk,),
        in_specs=[
            pl.BlockSpec((block, E), lambda i: (i, 0)),
            pl.BlockSpec((block, E), lambda i: (i, 0)),
            full(W1.shape), full((1, 248)),
            full(W2.shape), full((1, 128)),
            full(W3.shape), full((1, 64)),
            full(W4.shape), full((1, 32)),
            full(W5.shape), full((1, 1)),
        ],
        out_specs=pl.BlockSpec((block, 1), lambda i: (i, 0)),
        out_shape=jax.ShapeDtypeStruct((B, 1), jnp.float32),
        compiler_params=pltpu.CompilerParams(
            dimension_semantics=("arbitrary",)),
    )(uvec, bvec, W1, b1.reshape(1, -1), W2, b2.reshape(1, -1),
      W3, b3.reshape(1, -1), W4, b4.reshape(1, -1), W5, b5.reshape(1, -1))


@jax.jit
def kernel(inputs, user_table, book_table, W1, b1, W2, b2, W3, b3, W4, b4, W5, b5):
    idx = inputs.astype(jnp.int32)
    uidx = idx[:, 1].reshape(NW, NCHUNK, CHUNK)
    bidx = idx[:, 0].reshape(NW, NCHUNK, CHUNK)
    uvec, bvec = _sc_gather(uidx, bidx, user_table, book_table)
    return _mlp(uvec, bvec, W1, b1, W2, b2, W3, b3, W4, b4, W5, b5)
